# baseline (speedup 1.0000x reference)
import functools

import jax
import jax.numpy as jnp
from jax import lax
from jax.experimental import pallas as pl
from jax.experimental.pallas import tpu as pltpu
from jax.experimental.pallas import tpu_sc as plsc

D = 128
NC = 2
NS = 16
NW = NC * NS
CHUNK = 128
LANES = 16


def _mm_body(x_ref, w_ref, o_ref):
    o_ref[...] = jnp.dot(x_ref[...], w_ref[...],
                         preferred_element_type=jnp.float32)


def _matmul(x, W):
    M = x.shape[0]
    BM = 1000
    return pl.pallas_call(
        _mm_body,
        grid=(M // BM,),
        in_specs=[pl.BlockSpec((BM, D), lambda i: (i, 0)),
                  pl.BlockSpec((D, D), lambda i: (0, 0))],
        out_specs=pl.BlockSpec((BM, D), lambda i: (i, 0)),
        out_shape=jax.ShapeDtypeStruct((M, D), jnp.float32),
    )(x, W)


def _fin_body(p_ref, b_ref, o_ref):
    o_ref[...] = jnp.maximum(p_ref[0] + p_ref[1] + b_ref[...], 0.0)


def _finalize(partials, b, N):
    BM = 1000
    return pl.pallas_call(
        _fin_body,
        grid=(N // BM,),
        in_specs=[pl.BlockSpec((2, BM, D), lambda i: (0, i, 0)),
                  pl.BlockSpec((1, D), lambda i: (0, 0))],
        out_specs=pl.BlockSpec((BM, D), lambda i: (i, 0)),
        out_shape=jax.ShapeDtypeStruct((N, D), jnp.float32),
    )(partials, b.reshape(1, D))


NBUF = 2
SUPER = 8
SEDGES = SUPER * CHUNK


def _spmm(support, src, dst3, vals, e_per_w):
    N = support.shape[0]
    n_chunks = e_per_w // CHUNK
    n_super = n_chunks // SUPER
    rows_per_tile = ((N + NS - 1) // NS + 7) // 8 * 8
    NP = rows_per_tile * NS

    mesh = plsc.VectorSubcoreMesh(core_axis_name="c", subcore_axis_name="s")

    @functools.partial(
        pl.kernel,
        mesh=mesh,
        out_type=jax.ShapeDtypeStruct((NC, NP, D), jnp.float32),
        scratch_types=[
            pltpu.VMEM_SHARED((NP, D), jnp.float32),
            pltpu.VMEM((SEDGES,), jnp.int32),
            pltpu.VMEM((SUPER, 1, CHUNK), jnp.int32),
            pltpu.VMEM((SEDGES,), jnp.float32),
        ] + [pltpu.VMEM((CHUNK, D), jnp.float32) for _ in range(NBUF)]
          + [pltpu.SemaphoreType.DMA for _ in range(2 * NBUF + 3)],
    )
    def k(support_hbm, src_hbm, dst3_hbm, vals_hbm, out_hbm,
          acc, src_v, dst_v, vals_v, *bufs_sems):
        rows = bufs_sems[:NBUF]
        gsem = bufs_sems[NBUF:2 * NBUF]
        ssem = bufs_sems[2 * NBUF:3 * NBUF]
        isem = bufs_sems[3 * NBUF:]
        c = lax.axis_index("c")
        s = lax.axis_index("s")
        wid = c * NS + s

        def zrow(r, carry):
            for g in range(D // LANES):
                rows[0][r, pl.ds(g * LANES, LANES)] = jnp.zeros(
                    (LANES,), jnp.float32)
            return carry
        lax.fori_loop(0, CHUNK, zrow, 0)
        row0 = s * rows_per_tile
        nfull = rows_per_tile // CHUNK
        rem = rows_per_tile % CHUNK
        for i in range(nfull):
            pltpu.sync_copy(rows[0], acc.at[pl.ds(row0 + i * CHUNK, CHUNK)])
        if rem:
            pltpu.sync_copy(rows[0].at[pl.ds(0, rem)],
                            acc.at[pl.ds(row0 + nfull * CHUNK, rem)])
        plsc.subcore_barrier()

        cbase = wid * n_chunks

        def scatter_desc(p, ksel):
            return pltpu.make_async_copy(
                rows[p], acc.at[dst_v.at[ksel, 0]], ssem[p])

        def gather_start(kc, p):
            pltpu.async_copy(
                support_hbm.at[src_v.at[pl.ds(kc * CHUNK, CHUNK)]],
                rows[p], gsem[p])

        def gather_wait(kc, p):
            pltpu.make_async_copy(
                support_hbm.at[src_v.at[pl.ds(kc * CHUNK, CHUNK)]],
                rows[p], gsem[p]).wait()

        def sup_body(sup, carry):
            @pl.when(sup > 0)
            def _():
                for p in range(NBUF):
                    scatter_desc(p, 0).wait()
            ebase = (cbase + sup * SUPER) * CHUNK
            h1 = pltpu.async_copy(
                src_hbm.at[pl.ds(ebase, SEDGES)], src_v, isem[0])
            h2 = pltpu.async_copy(
                dst3_hbm.at[pl.ds(cbase + sup * SUPER, SUPER)], dst_v,
                isem[1])
            h3 = pltpu.async_copy(
                vals_hbm.at[pl.ds(ebase, SEDGES)], vals_v, isem[2])
            h1.wait()
            h2.wait()
            h3.wait()
            for t in range(min(NBUF - 1, SUPER)):
                gather_start(t, t)

            for kk in range(SUPER):
                p = kk % NBUF
                q = (kk + NBUF - 1) % NBUF
                gather_wait(kk, p)

                @plsc.parallel_loop(0, CHUNK // LANES, unroll=2)
                def scale(j16):
                    val16 = vals_v[pl.ds(kk * CHUNK + j16 * LANES, LANES)]
                    for l in range(LANES):
                        vj = lax.gather(
                            val16, jnp.full((LANES, 1), l, jnp.int32),
                            lax.GatherDimensionNumbers(
                                offset_dims=(), collapsed_slice_dims=(0,),
                                start_index_map=(0,)),
                            (1,),
                            mode=lax.GatherScatterMode.PROMISE_IN_BOUNDS)
                        j = j16 * LANES + l
                        for g in range(D // LANES):
                            rv = rows[p][j, pl.ds(g * LANES, LANES)]
                            rows[p][j, pl.ds(g * LANES, LANES)] = rv * vj

                pltpu.async_copy(rows[p], acc.at[dst_v.at[kk, 0]], ssem[p],
                                 add=True)
                if kk + NBUF - 1 < SUPER:
                    if kk >= 1:
                        scatter_desc(q, 0).wait()
                    gather_start(kk + NBUF - 1, q)
            return carry
        lax.fori_loop(0, n_super, sup_body, 0)
        for p in range(NBUF):
            scatter_desc(p, 0).wait()
        plsc.subcore_barrier()

        pltpu.sync_copy(acc.at[pl.ds(row0, rows_per_tile)],
                        out_hbm.at[c, pl.ds(row0, rows_per_tile)])

    return k(support, src, dst3, vals)


def kernel(x, edge_index, edge_vals, W, b):
    N = x.shape[0]
    E = edge_vals.shape[0]
    support = _matmul(x, W)

    e_per_w = ((E + NW - 1) // NW + SEDGES - 1) // SEDGES * SEDGES
    pad = e_per_w * NW - E
    src = edge_index[0]
    dst = edge_index[1]
    vals = edge_vals
    if pad:
        fill = jnp.arange(pad, dtype=jnp.int32) % N
        src = jnp.concatenate([src, fill])
        dst = jnp.concatenate([dst, fill])
        vals = jnp.concatenate([vals, jnp.zeros((pad,), vals.dtype)])
    dst3 = dst.reshape(-1, 1, CHUNK)

    partials = _spmm(support, src, dst3, vals, e_per_w)
    return _finalize(partials, b, N)


# speedup vs baseline: 1.0318x; 1.0318x over previous
import functools

import jax
import jax.numpy as jnp
from jax import lax
from jax.experimental import pallas as pl
from jax.experimental.pallas import tpu as pltpu
from jax.experimental.pallas import tpu_sc as plsc

D = 128
NC = 2
NS = 16
NW = NC * NS
CHUNK = 128
LANES = 16


def _mm_body(x_ref, w_ref, o_ref):
    o_ref[...] = jnp.dot(x_ref[...], w_ref[...],
                         preferred_element_type=jnp.float32)


def _matmul(x, W):
    M = x.shape[0]
    BM = 1000
    return pl.pallas_call(
        _mm_body,
        grid=(M // BM,),
        in_specs=[pl.BlockSpec((BM, D), lambda i: (i, 0)),
                  pl.BlockSpec((D, D), lambda i: (0, 0))],
        out_specs=pl.BlockSpec((BM, D), lambda i: (i, 0)),
        out_shape=jax.ShapeDtypeStruct((M, D), jnp.float32),
    )(x, W)


def _fin_body(p_ref, b_ref, o_ref):
    o_ref[...] = jnp.maximum(p_ref[0] + p_ref[1] + b_ref[...], 0.0)


def _finalize(partials, b, N):
    BM = 1000
    return pl.pallas_call(
        _fin_body,
        grid=(N // BM,),
        in_specs=[pl.BlockSpec((2, BM, D), lambda i: (0, i, 0)),
                  pl.BlockSpec((1, D), lambda i: (0, 0))],
        out_specs=pl.BlockSpec((BM, D), lambda i: (i, 0)),
        out_shape=jax.ShapeDtypeStruct((N, D), jnp.float32),
    )(partials, b.reshape(1, D))


NBUF = 2
SUB = 4
SUBR = CHUNK // SUB
SUPER = 8
SEDGES = SUPER * CHUNK


def _spmm(support, src, dst3, vals, e_per_w):
    N = support.shape[0]
    n_chunks = e_per_w // CHUNK
    n_super = n_chunks // SUPER
    rows_per_tile = ((N + NS - 1) // NS + 7) // 8 * 8
    NP = rows_per_tile * NS

    mesh = plsc.VectorSubcoreMesh(core_axis_name="c", subcore_axis_name="s")

    @functools.partial(
        pl.kernel,
        mesh=mesh,
        out_type=jax.ShapeDtypeStruct((NC, NP, D), jnp.float32),
        scratch_types=[
            pltpu.VMEM_SHARED((NP, D), jnp.float32),
            pltpu.VMEM((SEDGES,), jnp.int32),
            pltpu.VMEM((SUPER, 1, CHUNK), jnp.int32),
            pltpu.VMEM((SEDGES,), jnp.float32),
        ] + [pltpu.VMEM((CHUNK, D), jnp.float32) for _ in range(NBUF)]
          + [pltpu.SemaphoreType.DMA
             for _ in range(NBUF * SUB + NBUF + 3)],
    )
    def k(support_hbm, src_hbm, dst3_hbm, vals_hbm, out_hbm,
          acc, src_v, dst_v, vals_v, *bufs_sems):
        rows = bufs_sems[:NBUF]
        gsem = bufs_sems[NBUF:NBUF + NBUF * SUB]
        ssem = bufs_sems[NBUF + NBUF * SUB:NBUF + NBUF * SUB + NBUF]
        isem = bufs_sems[NBUF + NBUF * SUB + NBUF:]
        c = lax.axis_index("c")
        s = lax.axis_index("s")
        wid = c * NS + s

        def zrow(r, carry):
            for g in range(D // LANES):
                rows[0][r, pl.ds(g * LANES, LANES)] = jnp.zeros(
                    (LANES,), jnp.float32)
            return carry
        lax.fori_loop(0, CHUNK, zrow, 0)
        row0 = s * rows_per_tile
        nfull = rows_per_tile // CHUNK
        rem = rows_per_tile % CHUNK
        for i in range(nfull):
            pltpu.sync_copy(rows[0], acc.at[pl.ds(row0 + i * CHUNK, CHUNK)])
        if rem:
            pltpu.sync_copy(rows[0].at[pl.ds(0, rem)],
                            acc.at[pl.ds(row0 + nfull * CHUNK, rem)])
        plsc.subcore_barrier()

        cbase = wid * n_chunks

        def scatter_desc(p, ksel):
            return pltpu.make_async_copy(
                rows[p], acc.at[dst_v.at[ksel, 0]], ssem[p])

        def gather_start(kc, p):
            for u in range(SUB):
                pltpu.async_copy(
                    support_hbm.at[
                        src_v.at[pl.ds(kc * CHUNK + u * SUBR, SUBR)]],
                    rows[p].at[pl.ds(u * SUBR, SUBR)], gsem[p * SUB + u])

        def gather_wait(kc, p):
            for u in range(SUB):
                pltpu.make_async_copy(
                    support_hbm.at[
                        src_v.at[pl.ds(kc * CHUNK + u * SUBR, SUBR)]],
                    rows[p].at[pl.ds(u * SUBR, SUBR)],
                    gsem[p * SUB + u]).wait()

        def sup_body(sup, carry):
            @pl.when(sup > 0)
            def _():
                for p in range(NBUF):
                    scatter_desc(p, 0).wait()
            ebase = (cbase + sup * SUPER) * CHUNK
            h1 = pltpu.async_copy(
                src_hbm.at[pl.ds(ebase, SEDGES)], src_v, isem[0])
            h2 = pltpu.async_copy(
                dst3_hbm.at[pl.ds(cbase + sup * SUPER, SUPER)], dst_v,
                isem[1])
            h3 = pltpu.async_copy(
                vals_hbm.at[pl.ds(ebase, SEDGES)], vals_v, isem[2])
            h1.wait()
            h2.wait()
            h3.wait()
            for t in range(min(NBUF - 1, SUPER)):
                gather_start(t, t)

            for kk in range(SUPER):
                p = kk % NBUF
                q = (kk + NBUF - 1) % NBUF
                gather_wait(kk, p)

                def scale(j16, inner):
                    val16 = vals_v[pl.ds(kk * CHUNK + j16 * LANES, LANES)]
                    for l in range(LANES):
                        vj = lax.gather(
                            val16, jnp.full((LANES, 1), l, jnp.int32),
                            lax.GatherDimensionNumbers(
                                offset_dims=(), collapsed_slice_dims=(0,),
                                start_index_map=(0,)),
                            (1,),
                            mode=lax.GatherScatterMode.PROMISE_IN_BOUNDS)
                        j = j16 * LANES + l
                        for g in range(D // LANES):
                            rv = rows[p][j, pl.ds(g * LANES, LANES)]
                            rows[p][j, pl.ds(g * LANES, LANES)] = rv * vj
                    return inner
                lax.fori_loop(0, CHUNK // LANES, scale, 0)

                pltpu.async_copy(rows[p], acc.at[dst_v.at[kk, 0]],
                                 ssem[p], add=True)
                if kk + NBUF - 1 < SUPER:
                    if kk >= 1:
                        scatter_desc(q, 0).wait()
                    gather_start(kk + NBUF - 1, q)
            return carry
        lax.fori_loop(0, n_super, sup_body, 0)
        for p in range(NBUF):
            scatter_desc(p, 0).wait()
        plsc.subcore_barrier()

        pltpu.sync_copy(acc.at[pl.ds(row0, rows_per_tile)],
                        out_hbm.at[c, pl.ds(row0, rows_per_tile)])

    return k(support, src, dst3, vals)


def kernel(x, edge_index, edge_vals, W, b):
    N = x.shape[0]
    E = edge_vals.shape[0]
    support = _matmul(x, W)

    e_per_w = ((E + NW - 1) // NW + SEDGES - 1) // SEDGES * SEDGES
    pad = e_per_w * NW - E
    src = edge_index[0]
    dst = edge_index[1]
    vals = edge_vals
    if pad:
        fill = jnp.arange(pad, dtype=jnp.int32) % N
        src = jnp.concatenate([src, fill])
        dst = jnp.concatenate([dst, fill])
        vals = jnp.concatenate([vals, jnp.zeros((pad,), vals.dtype)])
    dst3 = dst.reshape(-1, 1, CHUNK)

    partials = _spmm(support, src, dst3, vals, e_per_w)
    return _finalize(partials, b, N)
